# Initial kernel scaffold; baseline (speedup 1.0000x reference)
import jax
import jax.numpy as jnp
from jax.experimental import pallas as pl


def kernel(hidden_states, deterministic, router_w, w0, w1, wo, shared_wi0, shared_wi1, shared_wo, shared_gate_w):
    raise NotImplementedError("write your pallas kernel here")


# speedup vs baseline: 2.3709x; 2.3709x over previous
import functools

import jax
import jax.numpy as jnp
from jax.experimental import pallas as pl
from jax.experimental.pallas import tpu as pltpu

B, S, D, E, F, K = 1, 2048, 1024, 8, 512, 2
T = B * S


def _moe_body(x_ref, rw_ref, w0_ref, w1_ref, wo_ref, swi0_ref, swi1_ref,
              swo_ref, sgw_ref, out_ref, loss_ref, combine_ref):
    e = pl.program_id(0)

    @pl.when(e == 0)
    def _prologue():
        x = x_ref[...]
        logits = jnp.dot(x, rw_ref[...], preferred_element_type=jnp.float32)
        probs = jax.nn.softmax(logits, axis=-1)
        i1 = jnp.argmax(probs, axis=-1)
        m1 = jnp.max(probs, axis=-1)
        iota = jax.lax.broadcasted_iota(jnp.int32, probs.shape, 1)
        oh1 = (iota == i1[:, None])
        masked = jnp.where(oh1, -jnp.inf, probs)
        i2 = jnp.argmax(masked, axis=-1)
        m2 = jnp.max(masked, axis=-1)
        oh2 = (iota == i2[:, None])
        denom = m1 + m2
        combine = (jnp.where(oh1, m1[:, None], 0.0)
                   + jnp.where(oh2, m2[:, None], 0.0)) / denom[:, None]
        combine_ref[...] = combine
        mask = (oh1 | oh2).astype(jnp.float32)
        frac_tokens = jnp.mean(mask, axis=0)
        mean_probs = jnp.mean(probs, axis=0)
        loss_ref[...] = (E * jnp.sum(frac_tokens * mean_probs)).reshape(1, 1)
        h0 = jnp.dot(x, swi0_ref[...], preferred_element_type=jnp.float32)
        h1 = jnp.dot(x, swi1_ref[...], preferred_element_type=jnp.float32)
        act = jax.nn.silu(h0) * h1
        shared = jnp.dot(act, swo_ref[...], preferred_element_type=jnp.float32)
        gate = jax.nn.sigmoid(
            jnp.dot(x, sgw_ref[...], preferred_element_type=jnp.float32))
        out_ref[...] = gate * shared

    x = x_ref[...]
    h0 = jnp.dot(x, w0_ref[0], preferred_element_type=jnp.float32)
    h1 = jnp.dot(x, w1_ref[0], preferred_element_type=jnp.float32)
    act = jax.nn.silu(h0) * h1
    y = jnp.dot(act, wo_ref[0], preferred_element_type=jnp.float32)
    comb = combine_ref[...]
    col = jax.lax.broadcasted_iota(jnp.int32, comb.shape, 1)
    cw = jnp.sum(jnp.where(col == e, comb, 0.0), axis=1)
    out_ref[...] += y * cw[:, None]


@jax.jit
def _moe(x, router_w, w0, w1, wo, shared_wi0, shared_wi1, shared_wo,
         shared_gate_w):
    out, loss = pl.pallas_call(
        _moe_body,
        grid=(E,),
        in_specs=[
            pl.BlockSpec((T, D), lambda e: (0, 0)),
            pl.BlockSpec((D, E), lambda e: (0, 0)),
            pl.BlockSpec((1, D, F), lambda e: (e, 0, 0)),
            pl.BlockSpec((1, D, F), lambda e: (e, 0, 0)),
            pl.BlockSpec((1, F, D), lambda e: (e, 0, 0)),
            pl.BlockSpec((D, F), lambda e: (0, 0)),
            pl.BlockSpec((D, F), lambda e: (0, 0)),
            pl.BlockSpec((F, D), lambda e: (0, 0)),
            pl.BlockSpec((D, 1), lambda e: (0, 0)),
        ],
        out_specs=[
            pl.BlockSpec((T, D), lambda e: (0, 0)),
            pl.BlockSpec((1, 1), lambda e: (0, 0)),
        ],
        out_shape=[
            jax.ShapeDtypeStruct((T, D), jnp.float32),
            jax.ShapeDtypeStruct((1, 1), jnp.float32),
        ],
        scratch_shapes=[pltpu.VMEM((T, E), jnp.float32)],
    )(x, router_w, w0, w1, wo, shared_wi0, shared_wi1, shared_wo,
      shared_gate_w)
    return out, loss[0, 0]


def kernel(hidden_states, deterministic, router_w, w0, w1, wo, shared_wi0,
           shared_wi1, shared_wo, shared_gate_w):
    x = hidden_states.reshape(-1, D)
    out, loss = _moe(x, router_w, w0, w1, wo, shared_wi0, shared_wi1,
                     shared_wo, shared_gate_w)
    return out.reshape(hidden_states.shape), loss


# speedup vs baseline: 2.3754x; 1.0019x over previous
import functools

import jax
import jax.numpy as jnp
from jax.experimental import pallas as pl
from jax.experimental.pallas import tpu as pltpu

B, S, D, E, F, K = 1, 2048, 1024, 8, 512, 2
T = B * S


def _moe_body(x_ref, rw_ref, w0_ref, w1_ref, wo_ref, swi0_ref, swi1_ref,
              swo_ref, sgw_ref, out_ref, loss_ref, combine_ref):
    e = pl.program_id(0)

    @pl.when(e == 0)
    def _prologue():
        x = x_ref[...]
        logits = jnp.dot(x, rw_ref[...], preferred_element_type=jnp.float32)
        probs = jax.nn.softmax(logits, axis=-1)
        i1 = jnp.argmax(probs, axis=-1)
        m1 = jnp.max(probs, axis=-1)
        iota = jax.lax.broadcasted_iota(jnp.int32, probs.shape, 1)
        oh1 = (iota == i1[:, None])
        masked = jnp.where(oh1, -jnp.inf, probs)
        i2 = jnp.argmax(masked, axis=-1)
        m2 = jnp.max(masked, axis=-1)
        oh2 = (iota == i2[:, None])
        denom = m1 + m2
        combine = (jnp.where(oh1, m1[:, None], 0.0)
                   + jnp.where(oh2, m2[:, None], 0.0)) / denom[:, None]
        combine_ref[...] = combine
        mask = (oh1 | oh2).astype(jnp.float32)
        frac_tokens = jnp.mean(mask, axis=0)
        mean_probs = jnp.mean(probs, axis=0)
        loss_ref[...] = (E * jnp.sum(frac_tokens * mean_probs)).reshape(1, 1)
        xb = x.astype(jnp.bfloat16)
        h0 = jnp.dot(xb, swi0_ref[...].astype(jnp.bfloat16),
                     preferred_element_type=jnp.float32)
        h1 = jnp.dot(xb, swi1_ref[...].astype(jnp.bfloat16),
                     preferred_element_type=jnp.float32)
        act = jax.nn.silu(h0) * h1
        shared = jnp.dot(act.astype(jnp.bfloat16),
                         swo_ref[...].astype(jnp.bfloat16),
                         preferred_element_type=jnp.float32)
        gate = jax.nn.sigmoid(
            jnp.dot(x, sgw_ref[...], preferred_element_type=jnp.float32))
        out_ref[...] = gate * shared

    x = x_ref[...].astype(jnp.bfloat16)
    h0 = jnp.dot(x, w0_ref[0].astype(jnp.bfloat16),
                 preferred_element_type=jnp.float32)
    h1 = jnp.dot(x, w1_ref[0].astype(jnp.bfloat16),
                 preferred_element_type=jnp.float32)
    act = jax.nn.silu(h0) * h1
    y = jnp.dot(act.astype(jnp.bfloat16), wo_ref[0].astype(jnp.bfloat16),
                preferred_element_type=jnp.float32)
    comb = combine_ref[...]
    col = jax.lax.broadcasted_iota(jnp.int32, comb.shape, 1)
    cw = jnp.sum(jnp.where(col == e, comb, 0.0), axis=1)
    out_ref[...] += y * cw[:, None]


@jax.jit
def _moe(x, router_w, w0, w1, wo, shared_wi0, shared_wi1, shared_wo,
         shared_gate_w):
    out, loss = pl.pallas_call(
        _moe_body,
        grid=(E,),
        in_specs=[
            pl.BlockSpec((T, D), lambda e: (0, 0)),
            pl.BlockSpec((D, E), lambda e: (0, 0)),
            pl.BlockSpec((1, D, F), lambda e: (e, 0, 0)),
            pl.BlockSpec((1, D, F), lambda e: (e, 0, 0)),
            pl.BlockSpec((1, F, D), lambda e: (e, 0, 0)),
            pl.BlockSpec((D, F), lambda e: (0, 0)),
            pl.BlockSpec((D, F), lambda e: (0, 0)),
            pl.BlockSpec((F, D), lambda e: (0, 0)),
            pl.BlockSpec((D, 1), lambda e: (0, 0)),
        ],
        out_specs=[
            pl.BlockSpec((T, D), lambda e: (0, 0)),
            pl.BlockSpec((1, 1), lambda e: (0, 0)),
        ],
        out_shape=[
            jax.ShapeDtypeStruct((T, D), jnp.float32),
            jax.ShapeDtypeStruct((1, 1), jnp.float32),
        ],
        scratch_shapes=[pltpu.VMEM((T, E), jnp.float32)],
    )(x, router_w, w0, w1, wo, shared_wi0, shared_wi1, shared_wo,
      shared_gate_w)
    return out, loss[0, 0]


def kernel(hidden_states, deterministic, router_w, w0, w1, wo, shared_wi0,
           shared_wi1, shared_wo, shared_gate_w):
    x = hidden_states.reshape(-1, D)
    out, loss = _moe(x, router_w, w0, w1, wo, shared_wi0, shared_wi1,
                     shared_wo, shared_gate_w)
    return out.reshape(hidden_states.shape), loss
